# baseline (speedup 1.0000x reference)
import functools

import jax
import jax.numpy as jnp
from jax import lax
from jax.experimental import pallas as pl
from jax.experimental.pallas import tpu as pltpu
from jax.experimental.pallas import tpu_sc as plsc

_CHUNK = 128
_VB = 2048


def _transpose_body(u_ref, i_ref, eye_ref, uo_ref, io_ref):
    eye = eye_ref[...]
    for src, dst in ((u_ref, uo_ref), (i_ref, io_ref)):
        xt = lax.dot_general(src[...], eye, (((0,), (0,)), ((), ())),
                             preferred_element_type=jnp.float32)
        dst[:, :64] = xt[:_VB // 2]
        dst[:, 64:] = xt[_VB // 2:]


def _make_transpose(V, D):
    grid = (V + _VB - 1) // _VB
    out_rows = grid * _VB // 2
    return pl.pallas_call(
        _transpose_body,
        grid=(grid,),
        in_specs=[
            pl.BlockSpec((D, _VB), lambda i: (0, i)),
            pl.BlockSpec((D, _VB), lambda i: (0, i)),
            pl.BlockSpec((D, D), lambda i: (0, 0)),
        ],
        out_specs=[
            pl.BlockSpec((_VB // 2, 2 * D), lambda i: (i, 0)),
            pl.BlockSpec((_VB // 2, 2 * D), lambda i: (i, 0)),
        ],
        out_shape=[
            jax.ShapeDtypeStruct((out_rows, 2 * D), jnp.float32),
            jax.ShapeDtypeStruct((out_rows, 2 * D), jnp.float32),
        ],
    )


def _make_sc_gather(B, W):
    info = plsc.get_sparse_core_info()
    NC, NS = info.num_cores, info.num_subcores
    NW = NC * NS
    assert B % (8 * NW) == 0
    b_per_w = B // NW
    n_chunks = b_per_w // _CHUNK
    assert n_chunks * _CHUNK == b_per_w

    mesh = plsc.VectorSubcoreMesh(core_axis_name="c", subcore_axis_name="s")

    @functools.partial(
        pl.kernel,
        mesh=mesh,
        compiler_params=pltpu.CompilerParams(skip_device_barrier=True),
        out_type=(
            jax.ShapeDtypeStruct((B, W), jnp.float32),
            jax.ShapeDtypeStruct((B, W), jnp.float32),
            jax.ShapeDtypeStruct((B, W), jnp.float32),
        ),
        scratch_types=[
            pltpu.VMEM((b_per_w,), jnp.int32),
            pltpu.VMEM((b_per_w,), jnp.int32),
            pltpu.VMEM((b_per_w,), jnp.int32),
            pltpu.VMEM((_CHUNK, W), jnp.float32),
            pltpu.VMEM((_CHUNK, W), jnp.float32),
            pltpu.VMEM((_CHUNK, W), jnp.float32),
            pltpu.SemaphoreType.DMA,
        ],
    )
    def k(user_hbm, pos_hbm, neg_hbm, ut_hbm, it_hbm,
          uout, pout, nout, uidx, pidx, nidx, ubuf, pbuf, nbuf, sem):
        wid = lax.axis_index("s") * NC + lax.axis_index("c")
        base = wid * b_per_w
        pltpu.sync_copy(user_hbm.at[pl.ds(base, b_per_w)], uidx)
        pltpu.sync_copy(pos_hbm.at[pl.ds(base, b_per_w)], pidx)
        pltpu.sync_copy(neg_hbm.at[pl.ds(base, b_per_w)], nidx)

        def remap(i, _):
            s = pl.ds(i * 16, 16)
            for ref in (uidx, pidx, nidx):
                ref[s] = lax.shift_right_logical(ref[s], 1)
            return _

        lax.fori_loop(0, b_per_w // 16, remap, 0)

        for j in range(n_chunks):
            sl = pl.ds(j * _CHUNK, _CHUNK)
            osl = pl.ds(base + j * _CHUNK, _CHUNK)
            cu = pltpu.async_copy(ut_hbm.at[uidx.at[sl]], ubuf, sem)
            cp = pltpu.async_copy(it_hbm.at[pidx.at[sl]], pbuf, sem)
            cn = pltpu.async_copy(it_hbm.at[nidx.at[sl]], nbuf, sem)
            cu.wait()
            pltpu.sync_copy(ubuf, uout.at[osl])
            cp.wait()
            pltpu.sync_copy(pbuf, pout.at[osl])
            cn.wait()
            pltpu.sync_copy(nbuf, nout.at[osl])

    return k


def kernel(user, pos, neg, user_table, item_table):
    B = user.shape[0]
    V, D = user_table.shape
    ut_pair = user_table.reshape(V // 2, 2 * D)
    it_pair = item_table.reshape(V // 2, 2 * D)
    k = _make_sc_gather(B, 2 * D)
    u, p, n = k(user.reshape(B), pos.reshape(B), neg.reshape(B),
                ut_pair, it_pair)

    def pick(pairs, idx):
        second = (idx & 1) == 1
        return jnp.where(second.reshape(B, 1), pairs[:, D:], pairs[:, :D])

    out = jnp.concatenate(
        [pick(u, user), pick(p, pos), pick(n, neg)], axis=-1)
    return out.reshape(B, 1, 3 * D)


# speedup vs baseline: 1.6185x; 1.6185x over previous
import functools

import jax
import jax.numpy as jnp
from jax import lax
from jax.experimental import pallas as pl
from jax.experimental.pallas import tpu as pltpu
from jax.experimental.pallas import tpu_sc as plsc

_CHUNK = 128
_VB = 2048


def _transpose_body(u_ref, i_ref, uo_ref, io_ref):
    ut = u_ref[...].T
    it = i_ref[...].T
    uo_ref[:, :64] = ut[:_VB // 2]
    io_ref[:, :64] = it[:_VB // 2]
    uo_ref[:, 64:] = ut[_VB // 2:]
    io_ref[:, 64:] = it[_VB // 2:]


def _make_transpose(V, D):
    grid = (V + _VB - 1) // _VB
    out_rows = grid * _VB // 2
    return pl.pallas_call(
        _transpose_body,
        grid=(grid,),
        in_specs=[
            pl.BlockSpec((D, _VB), lambda i: (0, i)),
            pl.BlockSpec((D, _VB), lambda i: (0, i)),
        ],
        out_specs=[
            pl.BlockSpec((_VB // 2, 2 * D), lambda i: (i, 0)),
            pl.BlockSpec((_VB // 2, 2 * D), lambda i: (i, 0)),
        ],
        out_shape=[
            jax.ShapeDtypeStruct((out_rows, 2 * D), jnp.float32),
            jax.ShapeDtypeStruct((out_rows, 2 * D), jnp.float32),
        ],
    )


def _make_sc_gather(B, W):
    info = plsc.get_sparse_core_info()
    NC, NS = info.num_cores, info.num_subcores
    NW = NC * NS
    assert B % (8 * NW) == 0
    b_per_w = B // NW
    n_chunks = b_per_w // _CHUNK
    assert n_chunks * _CHUNK == b_per_w

    mesh = plsc.VectorSubcoreMesh(core_axis_name="c", subcore_axis_name="s")

    @functools.partial(
        pl.kernel,
        mesh=mesh,
        out_type=(
            jax.ShapeDtypeStruct((B, W), jnp.float32),
            jax.ShapeDtypeStruct((B, W), jnp.float32),
            jax.ShapeDtypeStruct((B, W), jnp.float32),
        ),
        scratch_types=[
            pltpu.VMEM((b_per_w,), jnp.int32),
            pltpu.VMEM((b_per_w,), jnp.int32),
            pltpu.VMEM((b_per_w,), jnp.int32),
            pltpu.VMEM((_CHUNK, W), jnp.float32),
            pltpu.VMEM((_CHUNK, W), jnp.float32),
            pltpu.VMEM((_CHUNK, W), jnp.float32),
            pltpu.SemaphoreType.DMA,
        ],
    )
    def k(user_hbm, pos_hbm, neg_hbm, ut_hbm, it_hbm,
          uout, pout, nout, uidx, pidx, nidx, ubuf, pbuf, nbuf, sem):
        wid = lax.axis_index("s") * NC + lax.axis_index("c")
        base = wid * b_per_w
        pltpu.sync_copy(user_hbm.at[pl.ds(base, b_per_w)], uidx)
        pltpu.sync_copy(pos_hbm.at[pl.ds(base, b_per_w)], pidx)
        pltpu.sync_copy(neg_hbm.at[pl.ds(base, b_per_w)], nidx)

        def remap(i, _):
            s = pl.ds(i * 16, 16)
            for ref in (uidx, pidx, nidx):
                v = ref[s]
                ref[s] = jnp.bitwise_or(
                    lax.shift_left(lax.shift_right_logical(v, 11), 10),
                    jnp.bitwise_and(v, 1023))
            return _

        lax.fori_loop(0, b_per_w // 16, remap, 0)

        for j in range(n_chunks):
            sl = pl.ds(j * _CHUNK, _CHUNK)
            osl = pl.ds(base + j * _CHUNK, _CHUNK)
            cu = pltpu.async_copy(ut_hbm.at[uidx.at[sl]], ubuf, sem)
            cp = pltpu.async_copy(it_hbm.at[pidx.at[sl]], pbuf, sem)
            cn = pltpu.async_copy(it_hbm.at[nidx.at[sl]], nbuf, sem)
            cu.wait()
            pltpu.sync_copy(ubuf, uout.at[osl])
            cp.wait()
            pltpu.sync_copy(pbuf, pout.at[osl])
            cn.wait()
            pltpu.sync_copy(nbuf, nout.at[osl])

    return k


def kernel(user, pos, neg, user_table, item_table):
    B = user.shape[0]
    V, D = user_table.shape
    ut_pair, it_pair = _make_transpose(V, D)(user_table.T, item_table.T)
    k = _make_sc_gather(B, 2 * D)
    u, p, n = k(user.reshape(B), pos.reshape(B), neg.reshape(B),
                ut_pair, it_pair)

    def pick(pairs, idx):
        second = ((idx >> 10) & 1) == 1
        return jnp.where(second.reshape(B, 1), pairs[:, D:], pairs[:, :D])

    out = jnp.concatenate(
        [pick(u, user), pick(p, pos), pick(n, neg)], axis=-1)
    return out.reshape(B, 1, 3 * D)


# speedup vs baseline: 2.0142x; 1.2444x over previous
import functools

import jax
import jax.numpy as jnp
from jax import lax
from jax.experimental import pallas as pl
from jax.experimental.pallas import tpu as pltpu
from jax.experimental.pallas import tpu_sc as plsc

_CHUNK = 128
_VB = 4096
_SH = _VB.bit_length() - 1


def _transpose_body(u_ref, i_ref, uo_ref, io_ref):
    ut = u_ref[...].T
    it = i_ref[...].T
    uo_ref[:, :64] = ut[:_VB // 2]
    io_ref[:, :64] = it[:_VB // 2]
    uo_ref[:, 64:] = ut[_VB // 2:]
    io_ref[:, 64:] = it[_VB // 2:]


def _make_transpose(V, D):
    grid = (V + _VB - 1) // _VB
    out_rows = grid * _VB // 2
    return pl.pallas_call(
        _transpose_body,
        grid=(grid,),
        in_specs=[
            pl.BlockSpec((D, _VB), lambda i: (0, i)),
            pl.BlockSpec((D, _VB), lambda i: (0, i)),
        ],
        out_specs=[
            pl.BlockSpec((_VB // 2, 2 * D), lambda i: (i, 0)),
            pl.BlockSpec((_VB // 2, 2 * D), lambda i: (i, 0)),
        ],
        out_shape=[
            jax.ShapeDtypeStruct((out_rows, 2 * D), jnp.float32),
            jax.ShapeDtypeStruct((out_rows, 2 * D), jnp.float32),
        ],
    )


def _make_sc_gather(B, W):
    info = plsc.get_sparse_core_info()
    NC, NS = info.num_cores, info.num_subcores
    NW = NC * NS
    assert B % (8 * NW) == 0
    b_per_w = B // NW
    n_chunks = b_per_w // _CHUNK
    assert n_chunks * _CHUNK == b_per_w

    mesh = plsc.VectorSubcoreMesh(core_axis_name="c", subcore_axis_name="s")

    @functools.partial(
        pl.kernel,
        mesh=mesh,
        out_type=(
            jax.ShapeDtypeStruct((B, W), jnp.float32),
            jax.ShapeDtypeStruct((B, W), jnp.float32),
            jax.ShapeDtypeStruct((B, W), jnp.float32),
        ),
        scratch_types=[
            pltpu.VMEM((b_per_w,), jnp.int32),
            pltpu.VMEM((b_per_w,), jnp.int32),
            pltpu.VMEM((b_per_w,), jnp.int32),
            pltpu.VMEM((_CHUNK, W), jnp.float32),
            pltpu.VMEM((_CHUNK, W), jnp.float32),
            pltpu.VMEM((_CHUNK, W), jnp.float32),
            pltpu.SemaphoreType.DMA,
        ],
    )
    def k(user_hbm, pos_hbm, neg_hbm, ut_hbm, it_hbm,
          uout, pout, nout, uidx, pidx, nidx, ubuf, pbuf, nbuf, sem):
        wid = lax.axis_index("s") * NC + lax.axis_index("c")
        base = wid * b_per_w
        pltpu.sync_copy(user_hbm.at[pl.ds(base, b_per_w)], uidx)
        pltpu.sync_copy(pos_hbm.at[pl.ds(base, b_per_w)], pidx)
        pltpu.sync_copy(neg_hbm.at[pl.ds(base, b_per_w)], nidx)

        def remap(i, _):
            s = pl.ds(i * 16, 16)
            for ref in (uidx, pidx, nidx):
                v = ref[s]
                ref[s] = jnp.bitwise_or(
                    lax.shift_left(lax.shift_right_logical(v, _SH), _SH - 1),
                    jnp.bitwise_and(v, _VB // 2 - 1))
            return _

        lax.fori_loop(0, b_per_w // 16, remap, 0)

        for j in range(n_chunks):
            sl = pl.ds(j * _CHUNK, _CHUNK)
            osl = pl.ds(base + j * _CHUNK, _CHUNK)
            cu = pltpu.async_copy(ut_hbm.at[uidx.at[sl]], ubuf, sem)
            cp = pltpu.async_copy(it_hbm.at[pidx.at[sl]], pbuf, sem)
            cn = pltpu.async_copy(it_hbm.at[nidx.at[sl]], nbuf, sem)
            cu.wait()
            pltpu.sync_copy(ubuf, uout.at[osl])
            cp.wait()
            pltpu.sync_copy(pbuf, pout.at[osl])
            cn.wait()
            pltpu.sync_copy(nbuf, nout.at[osl])

    return k


def kernel(user, pos, neg, user_table, item_table):
    B = user.shape[0]
    V, D = user_table.shape
    ut_pair, it_pair = _make_transpose(V, D)(user_table.T, item_table.T)
    k = _make_sc_gather(B, 2 * D)
    u, p, n = k(user.reshape(B), pos.reshape(B), neg.reshape(B),
                ut_pair, it_pair)

    def pick(pairs, idx):
        second = ((idx >> (_SH - 1)) & 1) == 1
        return jnp.where(second.reshape(B, 1), pairs[:, D:], pairs[:, :D])

    out = jnp.concatenate(
        [pick(u, user), pick(p, pos), pick(n, neg)], axis=-1)
    return out.reshape(B, 1, 3 * D)


# speedup vs baseline: 2.3386x; 1.1611x over previous
import functools

import jax
import jax.numpy as jnp
from jax import lax
from jax.experimental import pallas as pl
from jax.experimental.pallas import tpu as pltpu
from jax.experimental.pallas import tpu_sc as plsc

_CHUNK = 128
_VB = 8192
_SH = _VB.bit_length() - 1


def _transpose_body(u_ref, i_ref, uo_ref, io_ref):
    ut = u_ref[...].T
    it = i_ref[...].T
    uo_ref[:, :64] = ut[:_VB // 2]
    io_ref[:, :64] = it[:_VB // 2]
    uo_ref[:, 64:] = ut[_VB // 2:]
    io_ref[:, 64:] = it[_VB // 2:]


def _make_transpose(V, D):
    grid = (V + _VB - 1) // _VB
    out_rows = grid * _VB // 2
    return pl.pallas_call(
        _transpose_body,
        grid=(grid,),
        in_specs=[
            pl.BlockSpec((D, _VB), lambda i: (0, i)),
            pl.BlockSpec((D, _VB), lambda i: (0, i)),
        ],
        out_specs=[
            pl.BlockSpec((_VB // 2, 2 * D), lambda i: (i, 0)),
            pl.BlockSpec((_VB // 2, 2 * D), lambda i: (i, 0)),
        ],
        out_shape=[
            jax.ShapeDtypeStruct((out_rows, 2 * D), jnp.float32),
            jax.ShapeDtypeStruct((out_rows, 2 * D), jnp.float32),
        ],
    )


def _make_sc_gather(B, W):
    info = plsc.get_sparse_core_info()
    NC, NS = info.num_cores, info.num_subcores
    NW = NC * NS
    assert B % (8 * NW) == 0
    b_per_w = B // NW
    n_chunks = b_per_w // _CHUNK
    assert n_chunks * _CHUNK == b_per_w

    mesh = plsc.VectorSubcoreMesh(core_axis_name="c", subcore_axis_name="s")

    @functools.partial(
        pl.kernel,
        mesh=mesh,
        out_type=(
            jax.ShapeDtypeStruct((B, W), jnp.float32),
            jax.ShapeDtypeStruct((B, W), jnp.float32),
            jax.ShapeDtypeStruct((B, W), jnp.float32),
        ),
        scratch_types=[
            pltpu.VMEM((b_per_w,), jnp.int32),
            pltpu.VMEM((b_per_w,), jnp.int32),
            pltpu.VMEM((b_per_w,), jnp.int32),
            pltpu.VMEM((_CHUNK, W), jnp.float32),
            pltpu.VMEM((_CHUNK, W), jnp.float32),
            pltpu.VMEM((_CHUNK, W), jnp.float32),
            pltpu.SemaphoreType.DMA,
        ],
    )
    def k(user_hbm, pos_hbm, neg_hbm, ut_hbm, it_hbm,
          uout, pout, nout, uidx, pidx, nidx, ubuf, pbuf, nbuf, sem):
        wid = lax.axis_index("s") * NC + lax.axis_index("c")
        base = wid * b_per_w
        pltpu.sync_copy(user_hbm.at[pl.ds(base, b_per_w)], uidx)
        pltpu.sync_copy(pos_hbm.at[pl.ds(base, b_per_w)], pidx)
        pltpu.sync_copy(neg_hbm.at[pl.ds(base, b_per_w)], nidx)

        def remap(i, _):
            s = pl.ds(i * 16, 16)
            for ref in (uidx, pidx, nidx):
                v = ref[s]
                ref[s] = jnp.bitwise_or(
                    lax.shift_left(lax.shift_right_logical(v, _SH), _SH - 1),
                    jnp.bitwise_and(v, _VB // 2 - 1))
            return _

        lax.fori_loop(0, b_per_w // 16, remap, 0)

        for j in range(n_chunks):
            sl = pl.ds(j * _CHUNK, _CHUNK)
            osl = pl.ds(base + j * _CHUNK, _CHUNK)
            cu = pltpu.async_copy(ut_hbm.at[uidx.at[sl]], ubuf, sem)
            cp = pltpu.async_copy(it_hbm.at[pidx.at[sl]], pbuf, sem)
            cn = pltpu.async_copy(it_hbm.at[nidx.at[sl]], nbuf, sem)
            cu.wait()
            pltpu.sync_copy(ubuf, uout.at[osl])
            cp.wait()
            pltpu.sync_copy(pbuf, pout.at[osl])
            cn.wait()
            pltpu.sync_copy(nbuf, nout.at[osl])

    return k


def kernel(user, pos, neg, user_table, item_table):
    B = user.shape[0]
    V, D = user_table.shape
    ut_pair, it_pair = _make_transpose(V, D)(user_table.T, item_table.T)
    k = _make_sc_gather(B, 2 * D)
    u, p, n = k(user.reshape(B), pos.reshape(B), neg.reshape(B),
                ut_pair, it_pair)

    def pick(pairs, idx):
        second = ((idx >> (_SH - 1)) & 1) == 1
        return jnp.where(second.reshape(B, 1), pairs[:, D:], pairs[:, :D])

    out = jnp.concatenate(
        [pick(u, user), pick(p, pos), pick(n, neg)], axis=-1)
    return out.reshape(B, 1, 3 * D)
